# Initial kernel scaffold; baseline (speedup 1.0000x reference)
import jax
import jax.numpy as jnp
from jax.experimental import pallas as pl


def kernel(x, position_ids, cos_cached, sin_cached):
    raise NotImplementedError("write your pallas kernel here")


# speedup vs baseline: 5.2016x; 5.2016x over previous
import functools

import jax
import jax.numpy as jnp
from jax import lax
from jax.experimental import pallas as pl
from jax.experimental.pallas import tpu as pltpu
from jax.experimental.pallas import tpu_sc as plsc

NC, NS = 2, 16
NW = NC * NS
CHUNK = 128


@functools.cache
def _build(n, dim):
    assert n % (NW * CHUNK) == 0
    n_per_w = n // NW
    n_chunks = n_per_w // CHUNK

    mesh = plsc.VectorSubcoreMesh(core_axis_name="c", subcore_axis_name="s")

    @functools.partial(
        pl.kernel,
        mesh=mesh,
        out_type=(
            jax.ShapeDtypeStruct((n, dim), jnp.float32),
            jax.ShapeDtypeStruct((n, dim), jnp.float32),
        ),
        scratch_types=[
            pltpu.VMEM((n_chunks, CHUNK), jnp.int32),
            pltpu.VMEM((2, CHUNK, dim), jnp.float32),
            pltpu.VMEM((2, CHUNK, dim), jnp.float32),
            pltpu.SemaphoreType.DMA,
            pltpu.SemaphoreType.DMA,
            pltpu.SemaphoreType.DMA,
            pltpu.SemaphoreType.DMA,
        ],
    )
    def k(idx_hbm, cos_hbm, sin_hbm, cos_out, sin_out,
          idx_v, cbuf, sbuf, sem_c0, sem_c1, sem_s0, sem_s1):
        wid = lax.axis_index("s") * NC + lax.axis_index("c")
        rbase = wid * n_per_w
        csems = (sem_c0, sem_c1)
        ssems = (sem_s0, sem_s1)

        pltpu.sync_copy(idx_hbm.at[pl.ds(wid * n_chunks, n_chunks)], idx_v)

        def fire(j, slot):
            hc = pltpu.async_copy(cos_hbm.at[idx_v.at[j]], cbuf.at[slot],
                                  csems[slot])
            hs = pltpu.async_copy(sin_hbm.at[idx_v.at[j]], sbuf.at[slot],
                                  ssems[slot])
            return hc, hs

        def drain(j, slot, handles):
            hc, hs = handles
            hc.wait()
            hs.wait()
            r0 = rbase + j * CHUNK
            pltpu.sync_copy(cbuf.at[slot], cos_out.at[pl.ds(r0, CHUNK)])
            pltpu.sync_copy(sbuf.at[slot], sin_out.at[pl.ds(r0, CHUNK)])

        handles = {0: fire(0, 0)}
        for j in range(1, n_chunks):
            handles[j] = fire(j, j % 2)
            drain(j - 1, (j - 1) % 2, handles.pop(j - 1))
        drain(n_chunks - 1, (n_chunks - 1) % 2, handles.pop(n_chunks - 1))

    return k


def kernel(x, position_ids, cos_cached, sin_cached):
    b, s = position_ids.shape
    n = b * s
    dim = cos_cached.shape[1]
    idx = position_ids.reshape(n // CHUNK, CHUNK)
    cos_flat, sin_flat = _build(n, dim)(idx, cos_cached, sin_cached)
    return (cos_flat.reshape(b, s, dim).astype(x.dtype),
            sin_flat.reshape(b, s, dim).astype(x.dtype))
